# baseline (speedup 1.0000x reference)
import functools
import jax
import jax.numpy as jnp
from jax import lax
from jax.experimental import pallas as pl
from jax.experimental.pallas import tpu as pltpu
from jax.experimental.pallas import tpu_sc as plsc

N = 10000
E = 320000
H = 64
G = 16
NODE_OUT = 3

NC = 2
NS = 16
NW = NC * NS

NPAD = 10240
EPAD = 327680
EW = EPAD // NW
CHUNK = 128
NCH = EW // CHUNK
NCH0 = 80
NCH1 = 80
ROWS_PT = NPAD // NS

BN = 1024
BE = 2048

_f32 = jnp.float32


def _encedge_body(ef, w1, b1, w2, b2, wc, bc, c0):
    h = jnp.maximum(jnp.dot(ef[...], w1[...], preferred_element_type=_f32) + b1[...], 0.0)
    he = jnp.dot(h, w2[...], preferred_element_type=_f32) + b2[...]
    c0[...] = jnp.dot(he, wc[...], preferred_element_type=_f32) + bc[...]


def _c_layer(ef, w1, b1, w2, b2, wc, bc):
    full = lambda s: pl.BlockSpec(s, lambda i: (0, 0))
    return pl.pallas_call(
        _encedge_body,
        grid=(EPAD // BE,),
        in_specs=[
            pl.BlockSpec((BE, 16), lambda i: (i, 0)),
            full((16, H)), full((1, H)), full((H, H)), full((1, H)),
            full((H, H)), full((1, H)),
        ],
        out_specs=pl.BlockSpec((BE, H), lambda i: (i, 0)),
        out_shape=jax.ShapeDtypeStruct((EPAD, H), _f32),
    )(ef, w1, b1, w2, b2, wc, bc)


def _mlp_body(x, w1, b1, w2, b2, o):
    h = jnp.maximum(jnp.dot(x[...], w1[...], preferred_element_type=_f32) + b1[...], 0.0)
    o[...] = jnp.dot(h, w2[...], preferred_element_type=_f32) + b2[...]


def _mlp_rows(x, w1, b1, w2, b2, dout):
    m, din = x.shape
    full = lambda s: pl.BlockSpec(s, lambda i: (0, 0))
    return pl.pallas_call(
        _mlp_body,
        grid=(m // BN,),
        in_specs=[
            pl.BlockSpec((BN, din), lambda i: (i, 0)),
            full((din, H)), full((1, H)), full((H, dout)), full((1, dout)),
        ],
        out_specs=pl.BlockSpec((BN, dout), lambda i: (i, 0)),
        out_shape=jax.ShapeDtypeStruct((m, dout), _f32),
    )(x, w1, b1, w2, b2)


def _encnode_body(nf, w1, b1, w2, b2, wab, hv, a, b):
    h = jnp.maximum(jnp.dot(nf[...], w1[...], preferred_element_type=_f32) + b1[...], 0.0)
    v = jnp.dot(h, w2[...], preferred_element_type=_f32) + b2[...]
    hv[...] = v
    ab = jnp.dot(v, wab[...], preferred_element_type=_f32)
    a[...] = ab[:, :H]
    b[...] = ab[:, H:]


def _enc_node(nf, w1, b1, w2, b2, wab):
    full = lambda s: pl.BlockSpec(s, lambda i: (0, 0))
    out = jax.ShapeDtypeStruct((NPAD, H), _f32)
    return pl.pallas_call(
        _encnode_body,
        grid=(NPAD // BN,),
        in_specs=[
            pl.BlockSpec((BN, 128), lambda i: (i, 0)),
            full((128, H)), full((1, H)), full((H, H)), full((1, H)),
            full((H, 2 * H)),
        ],
        out_specs=[pl.BlockSpec((BN, H), lambda i: (i, 0))] * 3,
        out_shape=[out, out, out],
    )(nf, w1, b1, w2, b2, wab)


def _update_body(s0, s1, c0, c1, hv, wm2, bm2, wu1a, wu1b, bu1, wu2, bu2,
                 wab, o, a, b):
    cnt = c0[:, :1] + c1[:, :1]
    s = s0[...] + s1[...]
    aggsum = jnp.dot(s, wm2[...], preferred_element_type=_f32) + cnt * bm2[...]
    agg = aggsum / jnp.clip(cnt, 1.0, None)
    h = jnp.maximum(
        jnp.dot(hv[...], wu1a[...], preferred_element_type=_f32)
        + jnp.dot(agg, wu1b[...], preferred_element_type=_f32) + bu1[...], 0.0)
    v = jnp.dot(h, wu2[...], preferred_element_type=_f32) + bu2[...]
    o[...] = v
    ab = jnp.dot(v, wab[...], preferred_element_type=_f32)
    a[...] = ab[:, :H]
    b[...] = ab[:, H:]


def _update(s0, s1, c0, c1, hv, wm2, bm2, wu1a, wu1b, bu1, wu2, bu2, wab):
    full = lambda s: pl.BlockSpec(s, lambda i: (0, 0))
    row = lambda d: pl.BlockSpec((BN, d), lambda i: (i, 0))
    out = jax.ShapeDtypeStruct((NPAD, H), _f32)
    return pl.pallas_call(
        _update_body,
        grid=(NPAD // BN,),
        in_specs=[
            row(H), row(H), row(16), row(16), row(H),
            full((H, H)), full((1, H)), full((H, H)), full((H, H)),
            full((1, H)), full((H, H)), full((1, H)), full((H, 2 * H)),
        ],
        out_specs=[row(H)] * 3,
        out_shape=[out, out, out],
    )(s0, s1, c0, c1, hv, wm2, bm2, wu1a, wu1b, bu1, wu2, bu2, wab)


def _readout_body(hv, bf, w1, b1, w2, b2, o, ssum, scnt):
    i = pl.program_id(0)

    @pl.when(i == 0)
    def _():
        ssum[...] = jnp.zeros_like(ssum)
        scnt[...] = jnp.zeros_like(scnt)

    gids = lax.broadcasted_iota(jnp.int32, (1, G), 1).astype(_f32)
    onehot = (bf[...] == gids).astype(_f32)
    dn = (((0,), (0,)), ((), ()))
    ssum[...] += lax.dot_general(onehot, hv[...], dn, preferred_element_type=_f32)
    scnt[...] += lax.dot_general(onehot, jnp.ones((BN, 1), _f32), dn,
                                 preferred_element_type=_f32)

    @pl.when(i == pl.num_programs(0) - 1)
    def _():
        pooled = ssum[...] / jnp.clip(scnt[...], 1.0, None)
        h = jnp.maximum(jnp.dot(pooled, w1[...], preferred_element_type=_f32) + b1[...], 0.0)
        o[...] = jnp.dot(h, w2[...], preferred_element_type=_f32) + b2[...]


def _readout(hv, bf, w1, b1, w2, b2):
    full = lambda s: pl.BlockSpec(s, lambda i: (0, 0))
    return pl.pallas_call(
        _readout_body,
        grid=(NPAD // BN,),
        in_specs=[
            pl.BlockSpec((BN, H), lambda i: (i, 0)),
            pl.BlockSpec((BN, 1), lambda i: (i, 0)),
            full((H, H)), full((1, H)), full((H, 1)), full((1, 1)),
        ],
        out_specs=full((G, 1)),
        out_shape=jax.ShapeDtypeStruct((G, 1), _f32),
        scratch_shapes=[pltpu.VMEM((G, H), _f32), pltpu.VMEM((G, 1), _f32)],
    )(hv, bf, w1, b1, w2, b2)


_MESH = dict(core_axis_name="c", subcore_axis_name="s")


def _sc_counts(dst_p, ones_rows, zeros_cnt):
    CHC = 256
    nch = EW // CHC

    @functools.partial(
        pl.kernel,
        out_type=jax.ShapeDtypeStruct((NC, NPAD, 16), _f32),
        mesh=plsc.VectorSubcoreMesh(**_MESH),
        compiler_params=pltpu.CompilerParams(use_tc_tiling_on_sc=False),
        scratch_types=[
            pltpu.VMEM((256,), jnp.int32),
            pltpu.VMEM((256, 16), _f32),
            pltpu.VMEM_SHARED((NPAD, 16), _f32),
        ],
    )
    def k(dst_hbm, ones_hbm, zero_hbm, out_hbm, idx_v, ones_v, cnt_sh):
        cid = lax.axis_index("c")
        sid = lax.axis_index("s")
        wid = sid * NC + cid

        @pl.when(sid == 0)
        def _():
            pltpu.sync_copy(zero_hbm, cnt_sh)
        pltpu.sync_copy(ones_hbm, ones_v)
        plsc.subcore_barrier()

        def body(j, _):
            base = wid * EW + j * CHC
            pltpu.sync_copy(dst_hbm.at[pl.ds(base, CHC)], idx_v)
            pltpu.sync_copy(ones_v, cnt_sh.at[idx_v], add=True)
            return 0
        lax.fori_loop(0, nch, body, 0)
        plsc.subcore_barrier()
        pltpu.sync_copy(cnt_sh.at[pl.ds(sid * ROWS_PT, ROWS_PT)],
                        out_hbm.at[cid, pl.ds(sid * ROWS_PT, ROWS_PT)])

    return k(dst_p, ones_rows, zeros_cnt)


def _sc_edge(a_t, b_t, c_e, src_p, dst_p, zeros_s):
    @functools.partial(
        pl.kernel,
        out_type=jax.ShapeDtypeStruct((NC, NPAD, H), _f32),
        mesh=plsc.VectorSubcoreMesh(**_MESH),
        compiler_params=pltpu.CompilerParams(use_tc_tiling_on_sc=False),
        scratch_types=[
            pltpu.VMEM((CHUNK,), jnp.int32),
            pltpu.VMEM((CHUNK,), jnp.int32),
            pltpu.VMEM((CHUNK,), jnp.int32),
            pltpu.VMEM((CHUNK,), jnp.int32),
            pltpu.VMEM((CHUNK,), jnp.int32),
            pltpu.VMEM((CHUNK,), jnp.int32),
            pltpu.VMEM((CHUNK, H), _f32),
            pltpu.VMEM((CHUNK, H), _f32),
            pltpu.VMEM((CHUNK, H), _f32),
            pltpu.VMEM((CHUNK, H), _f32),
            pltpu.VMEM((CHUNK, H), _f32),
            pltpu.VMEM((CHUNK, H), _f32),
            pltpu.VMEM((CHUNK, H), _f32),
            pltpu.VMEM((CHUNK, H), _f32),
            pltpu.SemaphoreType.DMA,
            pltpu.SemaphoreType.DMA,
            pltpu.SemaphoreType.DMA,
            pltpu.SemaphoreType.DMA,
            pltpu.SemaphoreType.DMA,
            pltpu.SemaphoreType.DMA,
            pltpu.VMEM_SHARED((NPAD, H), _f32),
        ],
    )
    def k(a_hbm, b_hbm, c_hbm, src_hbm, dst_hbm, zero_hbm, out_hbm,
          is0, is1, id0, id1, ic0, ic1, ba0, ba1, bb0, bb1, bc0, bc1,
          bh0, bh1, si0, si1, sg0, sg1, ss0, ss1, s_sh):
        IS, ID, IC = [is0, is1], [id0, id1], [ic0, ic1]
        BA, BB, BC, BH = [ba0, ba1], [bb0, bb1], [bc0, bc1], [bh0, bh1]
        SI, SG, SS = [si0, si1], [sg0, sg1], [ss0, ss1]

        cid = lax.axis_index("c")
        sid = lax.axis_index("s")
        nch = jnp.where(cid == 0, NCH0, NCH1)
        base0 = CHUNK * jnp.where(cid == 0, sid * NCH0, 16 * NCH0 + sid * NCH1)
        tile_rows = pl.ds(sid * ROWS_PT, ROWS_PT)
        pltpu.sync_copy(zero_hbm, s_sh.at[tile_rows])
        plsc.subcore_barrier()

        def start_idx(s, j):
            base = base0 + j * CHUNK
            pltpu.async_copy(src_hbm.at[pl.ds(base, CHUNK)], IS[s], SI[s])
            pltpu.async_copy(dst_hbm.at[pl.ds(base, CHUNK)], ID[s], SI[s])
            pltpu.async_copy(c_hbm.at[pl.ds(base, CHUNK)], BC[s], SI[s])

        def wait_idx(s):
            pltpu.make_async_copy(src_hbm.at[pl.ds(0, CHUNK)], IS[s], SI[s]).wait()
            pltpu.make_async_copy(dst_hbm.at[pl.ds(0, CHUNK)], ID[s], SI[s]).wait()
            pltpu.make_async_copy(c_hbm.at[pl.ds(0, CHUNK)], BC[s], SI[s]).wait()

        def start_gather(s):
            pltpu.async_copy(a_hbm.at[IS[s]], BA[s], SG[s])
            pltpu.async_copy(b_hbm.at[ID[s]], BB[s], SG[s])

        def wait_gather(s):
            pltpu.make_async_copy(a_hbm.at[IS[s]], BA[s], SG[s]).wait()
            pltpu.make_async_copy(b_hbm.at[ID[s]], BB[s], SG[s]).wait()

        def start_scatter(s):
            pltpu.async_copy(BH[s], s_sh.at[IC[s]], SS[s], add=True)

        def wait_scatter(s):
            pltpu.make_async_copy(BH[s], s_sh.at[IC[s]], SS[s]).wait()

        def compute(s):
            a_, b_, c_, h_ = BA[s], BB[s], BC[s], BH[s]

            @plsc.parallel_loop(0, CHUNK, unroll=4)
            def _(r):
                for c4 in range(H // 16):
                    sl = pl.ds(c4 * 16, 16)
                    h_[r, sl] = jnp.maximum(a_[r, sl] + b_[r, sl] + c_[r, sl], 0.0)

        def phase(j, s):
            o = 1 - s

            @pl.when(jnp.logical_and(j >= 3, j <= nch + 1))
            def _():
                wait_scatter(o)

            @pl.when(j < nch)
            def _():
                wait_idx(s)
                start_gather(s)

            @pl.when(jnp.logical_and(j >= 1, j <= nch))
            def _():
                wait_gather(o)
                for q in range(CHUNK // 16):
                    qs = pl.ds(q * 16, 16)
                    IC[o][qs] = ID[o][qs]
                compute(o)
                start_scatter(o)

            @pl.when(j + 1 < nch)
            def _():
                start_idx(o, j + 1)

        start_idx(0, 0)

        def pair(i, _):
            j = 2 * i
            phase(j, 0)
            phase(j + 1, 1)
            return 0
        lax.fori_loop(0, (max(NCH0, NCH1) + 2 + 1) // 2, pair, 0)

        @pl.when(nch % 2 == 1)
        def _():
            wait_scatter(0)

        @pl.when(nch % 2 == 0)
        def _():
            wait_scatter(1)

        plsc.subcore_barrier()
        pltpu.sync_copy(s_sh.at[tile_rows], out_hbm.at[cid, tile_rows])

    return k(a_t, b_t, c_e, src_p, dst_p, zeros_s)


def kernel(node_features, edge_features, edge_index, batch, params):
    p = params
    src_p = jnp.pad(edge_index[0], (0, EPAD - E))
    dst_p = jnp.pad(edge_index[1], (0, EPAD - E), constant_values=N)
    nf = jnp.pad(node_features, ((0, NPAD - N), (0, 0)))
    ef = jnp.pad(edge_features, ((0, EPAD - E), (0, 0)))
    bf = jnp.pad(batch.astype(_f32), (0, NPAD - N),
                 constant_values=float(G)).reshape(NPAD, 1)

    enW1, enb1, enW2, enb2 = p['enc_node']
    eeW1, eeb1, eeW2, eeb2 = p['enc_edge']
    wc = jnp.concatenate([lp['msg'][0][2 * H:3 * H, :] for lp in p['layers']], axis=1)
    bc = jnp.concatenate([lp['msg'][1] for lp in p['layers']]).reshape(1, 4 * H)

    def wab_of(l):
        mW1 = p['layers'][l]['msg'][0]
        return jnp.concatenate([mW1[:H], mW1[H:2 * H]], axis=1)

    hv, a_t, b_t = _enc_node(nf, enW1, enb1.reshape(1, H), enW2,
                             enb2.reshape(1, H), wab_of(0))

    ones_rows = jnp.zeros((256, 16), _f32).at[:, 0].set(1.0)
    cnt2 = _sc_counts(dst_p, ones_rows, jnp.zeros((NPAD, 16), _f32))
    cnt0, cnt1 = cnt2[0], cnt2[1]

    zeros_s = jnp.zeros((ROWS_PT, H), _f32)
    for l, lp in enumerate(p['layers']):
        mW1, mb1, mW2, mb2 = lp['msg']
        uW1, ub1, uW2, ub2 = lp['upd']
        c_l = _c_layer(ef, eeW1, eeb1.reshape(1, H), eeW2, eeb2.reshape(1, H),
                       wc[:, l * H:(l + 1) * H], bc[:, l * H:(l + 1) * H])
        s2 = _sc_edge(a_t, b_t, c_l, src_p, dst_p, zeros_s)
        wab_next = wab_of(l + 1) if l + 1 < len(p['layers']) else jnp.zeros((H, 2 * H), _f32)
        hv, a_t, b_t = _update(s2[0], s2[1], cnt0, cnt1, hv, mW2,
                               mb2.reshape(1, H), uW1[:H], uW1[H:],
                               ub1.reshape(1, H), uW2, ub2.reshape(1, H), wab_next)

    dW1, db1, dW2, db2 = p['dec']
    node_pred = _mlp_rows(hv, dW1, db1.reshape(1, H), dW2,
                          db2.reshape(1, NODE_OUT), NODE_OUT)[:N]

    rW1, rb1, rW2, rb2 = p['ro']
    global_pred = _readout(hv, bf, rW1, rb1.reshape(1, H), rW2, rb2.reshape(1, 1))
    return node_pred, global_pred


# speedup vs baseline: 1.1622x; 1.1622x over previous
import functools
import jax
import jax.numpy as jnp
from jax import lax
from jax.experimental import pallas as pl
from jax.experimental.pallas import tpu as pltpu
from jax.experimental.pallas import tpu_sc as plsc

N = 10000
E = 320000
H = 64
G = 16
NODE_OUT = 3

NC = 2
NS = 16
NW = NC * NS

NPAD = 10240
EPAD = 327680
EW = EPAD // NW
CHUNK = 128
NCH = EW // CHUNK
NCH0 = 80
NCH1 = 80
ROWS_PT = NPAD // NS

BN = 1024
BE = 2048

_f32 = jnp.float32


def _encedge_body(ef, w1, b1, w2, b2, wc, bc, c0, c1, c2, c3):
    h = jnp.maximum(jnp.dot(ef[...], w1[...], preferred_element_type=_f32) + b1[...], 0.0)
    he = jnp.dot(h, w2[...], preferred_element_type=_f32) + b2[...]
    cf = jnp.dot(he, wc[...], preferred_element_type=_f32) + bc[...]
    c0[...] = cf[:, 0:64]
    c1[...] = cf[:, 64:128]
    c2[...] = cf[:, 128:192]
    c3[...] = cf[:, 192:256]


def _enc_edge(ef, w1, b1, w2, b2, wc, bc):
    full = lambda s: pl.BlockSpec(s, lambda i: (0, 0))
    out = jax.ShapeDtypeStruct((EPAD, H), _f32)
    return pl.pallas_call(
        _encedge_body,
        grid=(EPAD // BE,),
        in_specs=[
            pl.BlockSpec((BE, 16), lambda i: (i, 0)),
            full((16, H)), full((1, H)), full((H, H)), full((1, H)),
            full((H, 4 * H)), full((1, 4 * H)),
        ],
        out_specs=[pl.BlockSpec((BE, H), lambda i: (i, 0))] * 4,
        out_shape=[out, out, out, out],
    )(ef, w1, b1, w2, b2, wc, bc)


def _mlp_body(x, w1, b1, w2, b2, o):
    h = jnp.maximum(jnp.dot(x[...], w1[...], preferred_element_type=_f32) + b1[...], 0.0)
    o[...] = jnp.dot(h, w2[...], preferred_element_type=_f32) + b2[...]


def _mlp_rows(x, w1, b1, w2, b2, dout):
    m, din = x.shape
    full = lambda s: pl.BlockSpec(s, lambda i: (0, 0))
    return pl.pallas_call(
        _mlp_body,
        grid=(m // BN,),
        in_specs=[
            pl.BlockSpec((BN, din), lambda i: (i, 0)),
            full((din, H)), full((1, H)), full((H, dout)), full((1, dout)),
        ],
        out_specs=pl.BlockSpec((BN, dout), lambda i: (i, 0)),
        out_shape=jax.ShapeDtypeStruct((m, dout), _f32),
    )(x, w1, b1, w2, b2)


def _ab_body(hv, wab, a, b):
    ab = jnp.dot(hv[...], wab[...], preferred_element_type=_f32)
    a[...] = ab[:, :H]
    b[...] = ab[:, H:]


def _ab(hv, wab):
    out = jax.ShapeDtypeStruct((NPAD, H), _f32)
    return pl.pallas_call(
        _ab_body,
        grid=(NPAD // BN,),
        in_specs=[
            pl.BlockSpec((BN, H), lambda i: (i, 0)),
            pl.BlockSpec((H, 2 * H), lambda i: (0, 0)),
        ],
        out_specs=[pl.BlockSpec((BN, H), lambda i: (i, 0))] * 2,
        out_shape=[out, out],
    )(hv, wab)


def _update_body(s0, s1, c0, c1, hv, wm2, bm2, wu1a, wu1b, bu1, wu2, bu2, o):
    cnt = c0[:, :1] + c1[:, :1]
    s = s0[...] + s1[...]
    aggsum = jnp.dot(s, wm2[...], preferred_element_type=_f32) + cnt * bm2[...]
    agg = aggsum / jnp.clip(cnt, 1.0, None)
    h = jnp.maximum(
        jnp.dot(hv[...], wu1a[...], preferred_element_type=_f32)
        + jnp.dot(agg, wu1b[...], preferred_element_type=_f32) + bu1[...], 0.0)
    o[...] = jnp.dot(h, wu2[...], preferred_element_type=_f32) + bu2[...]


def _update(s0, s1, c0, c1, hv, wm2, bm2, wu1a, wu1b, bu1, wu2, bu2):
    full = lambda s: pl.BlockSpec(s, lambda i: (0, 0))
    row = lambda d: pl.BlockSpec((BN, d), lambda i: (i, 0))
    return pl.pallas_call(
        _update_body,
        grid=(NPAD // BN,),
        in_specs=[
            row(H), row(H), row(16), row(16), row(H),
            full((H, H)), full((1, H)), full((H, H)), full((H, H)),
            full((1, H)), full((H, H)), full((1, H)),
        ],
        out_specs=row(H),
        out_shape=jax.ShapeDtypeStruct((NPAD, H), _f32),
    )(s0, s1, c0, c1, hv, wm2, bm2, wu1a, wu1b, bu1, wu2, bu2)


def _readout_body(hv, bf, w1, b1, w2, b2, o, ssum, scnt):
    i = pl.program_id(0)

    @pl.when(i == 0)
    def _():
        ssum[...] = jnp.zeros_like(ssum)
        scnt[...] = jnp.zeros_like(scnt)

    gids = lax.broadcasted_iota(jnp.int32, (1, G), 1).astype(_f32)
    onehot = (bf[...] == gids).astype(_f32)
    dn = (((0,), (0,)), ((), ()))
    ssum[...] += lax.dot_general(onehot, hv[...], dn, preferred_element_type=_f32)
    scnt[...] += lax.dot_general(onehot, jnp.ones((BN, 1), _f32), dn,
                                 preferred_element_type=_f32)

    @pl.when(i == pl.num_programs(0) - 1)
    def _():
        pooled = ssum[...] / jnp.clip(scnt[...], 1.0, None)
        h = jnp.maximum(jnp.dot(pooled, w1[...], preferred_element_type=_f32) + b1[...], 0.0)
        o[...] = jnp.dot(h, w2[...], preferred_element_type=_f32) + b2[...]


def _readout(hv, bf, w1, b1, w2, b2):
    full = lambda s: pl.BlockSpec(s, lambda i: (0, 0))
    return pl.pallas_call(
        _readout_body,
        grid=(NPAD // BN,),
        in_specs=[
            pl.BlockSpec((BN, H), lambda i: (i, 0)),
            pl.BlockSpec((BN, 1), lambda i: (i, 0)),
            full((H, H)), full((1, H)), full((H, 1)), full((1, 1)),
        ],
        out_specs=full((G, 1)),
        out_shape=jax.ShapeDtypeStruct((G, 1), _f32),
        scratch_shapes=[pltpu.VMEM((G, H), _f32), pltpu.VMEM((G, 1), _f32)],
    )(hv, bf, w1, b1, w2, b2)


_MESH = dict(core_axis_name="c", subcore_axis_name="s")


def _sc_counts(dst_p, ones_rows, zeros_cnt):
    CHC = 256
    nch = EW // CHC

    @functools.partial(
        pl.kernel,
        out_type=jax.ShapeDtypeStruct((NC, NPAD, 16), _f32),
        mesh=plsc.VectorSubcoreMesh(**_MESH),
        compiler_params=pltpu.CompilerParams(use_tc_tiling_on_sc=False),
        scratch_types=[
            pltpu.VMEM((256,), jnp.int32),
            pltpu.VMEM((256, 16), _f32),
            pltpu.VMEM_SHARED((NPAD, 16), _f32),
        ],
    )
    def k(dst_hbm, ones_hbm, zero_hbm, out_hbm, idx_v, ones_v, cnt_sh):
        cid = lax.axis_index("c")
        sid = lax.axis_index("s")
        wid = sid * NC + cid

        @pl.when(sid == 0)
        def _():
            pltpu.sync_copy(zero_hbm, cnt_sh)
        pltpu.sync_copy(ones_hbm, ones_v)
        plsc.subcore_barrier()

        def body(j, _):
            base = wid * EW + j * CHC
            pltpu.sync_copy(dst_hbm.at[pl.ds(base, CHC)], idx_v)
            pltpu.sync_copy(ones_v, cnt_sh.at[idx_v], add=True)
            return 0
        lax.fori_loop(0, nch, body, 0)
        plsc.subcore_barrier()
        pltpu.sync_copy(cnt_sh.at[pl.ds(sid * ROWS_PT, ROWS_PT)],
                        out_hbm.at[cid, pl.ds(sid * ROWS_PT, ROWS_PT)])

    return k(dst_p, ones_rows, zeros_cnt)


def _sc_edge(a_t, b_t, c_e, src_p, dst_p, zeros_s):
    @functools.partial(
        pl.kernel,
        out_type=jax.ShapeDtypeStruct((NC, NPAD, H), _f32),
        mesh=plsc.VectorSubcoreMesh(**_MESH),
        compiler_params=pltpu.CompilerParams(use_tc_tiling_on_sc=False),
        scratch_types=[
            pltpu.VMEM((CHUNK,), jnp.int32),
            pltpu.VMEM((CHUNK,), jnp.int32),
            pltpu.VMEM((CHUNK,), jnp.int32),
            pltpu.VMEM((CHUNK,), jnp.int32),
            pltpu.VMEM((CHUNK,), jnp.int32),
            pltpu.VMEM((CHUNK,), jnp.int32),
            pltpu.VMEM((CHUNK, H), _f32),
            pltpu.VMEM((CHUNK, H), _f32),
            pltpu.VMEM((CHUNK, H), _f32),
            pltpu.VMEM((CHUNK, H), _f32),
            pltpu.VMEM((CHUNK, H), _f32),
            pltpu.VMEM((CHUNK, H), _f32),
            pltpu.VMEM((CHUNK, H), _f32),
            pltpu.VMEM((CHUNK, H), _f32),
            pltpu.SemaphoreType.DMA,
            pltpu.SemaphoreType.DMA,
            pltpu.SemaphoreType.DMA,
            pltpu.SemaphoreType.DMA,
            pltpu.SemaphoreType.DMA,
            pltpu.SemaphoreType.DMA,
            pltpu.VMEM_SHARED((NPAD, H), _f32),
        ],
    )
    def k(a_hbm, b_hbm, c_hbm, src_hbm, dst_hbm, zero_hbm, out_hbm,
          is0, is1, id0, id1, ic0, ic1, ba0, ba1, bb0, bb1, bc0, bc1,
          bh0, bh1, si0, si1, sg0, sg1, ss0, ss1, s_sh):
        IS, ID, IC = [is0, is1], [id0, id1], [ic0, ic1]
        BA, BB, BC, BH = [ba0, ba1], [bb0, bb1], [bc0, bc1], [bh0, bh1]
        SI, SG, SS = [si0, si1], [sg0, sg1], [ss0, ss1]

        cid = lax.axis_index("c")
        sid = lax.axis_index("s")
        wid = sid * NC + cid
        base0 = wid * EW
        tile_rows = pl.ds(sid * ROWS_PT, ROWS_PT)

        @pl.when(sid == 0)
        def _():
            pltpu.sync_copy(zero_hbm, s_sh)
        plsc.subcore_barrier()

        def start_idx(s, j):
            base = base0 + j * CHUNK
            pltpu.async_copy(src_hbm.at[pl.ds(base, CHUNK)], IS[s], SI[s])
            pltpu.async_copy(dst_hbm.at[pl.ds(base, CHUNK)], ID[s], SI[s])
            pltpu.async_copy(c_hbm.at[pl.ds(base, CHUNK)], BC[s], SI[s])

        def wait_idx(s):
            pltpu.make_async_copy(src_hbm.at[pl.ds(0, CHUNK)], IS[s], SI[s]).wait()
            pltpu.make_async_copy(dst_hbm.at[pl.ds(0, CHUNK)], ID[s], SI[s]).wait()
            pltpu.make_async_copy(c_hbm.at[pl.ds(0, CHUNK)], BC[s], SI[s]).wait()

        def start_gather(s):
            pltpu.async_copy(a_hbm.at[IS[s]], BA[s], SG[s])
            pltpu.async_copy(b_hbm.at[ID[s]], BB[s], SG[s])

        def wait_gather(s):
            pltpu.make_async_copy(a_hbm.at[IS[s]], BA[s], SG[s]).wait()
            pltpu.make_async_copy(b_hbm.at[ID[s]], BB[s], SG[s]).wait()

        def start_scatter(s):
            pltpu.async_copy(BH[s], s_sh.at[IC[s]], SS[s], add=True)

        def wait_scatter(s):
            pltpu.make_async_copy(BH[s], s_sh.at[IC[s]], SS[s]).wait()

        def compute(s):
            a_, b_, c_, h_ = BA[s], BB[s], BC[s], BH[s]

            @plsc.parallel_loop(0, CHUNK, unroll=4)
            def _(r):
                for c4 in range(H // 16):
                    sl = pl.ds(c4 * 16, 16)
                    h_[r, sl] = jnp.maximum(a_[r, sl] + b_[r, sl] + c_[r, sl], 0.0)

        def phase(j, s):
            o = 1 - s

            @pl.when(j >= 3)
            def _():
                wait_scatter(o)

            @pl.when(j < NCH)
            def _():
                wait_idx(s)
                start_gather(s)

            @pl.when(jnp.logical_and(j >= 1, j <= NCH))
            def _():
                wait_gather(o)
                for q in range(CHUNK // 16):
                    qs = pl.ds(q * 16, 16)
                    IC[o][qs] = ID[o][qs]
                compute(o)
                start_scatter(o)

            @pl.when(j + 1 < NCH)
            def _():
                start_idx(o, j + 1)

        start_idx(0, 0)

        def pair(i, _):
            j = 2 * i
            phase(j, 0)
            phase(j + 1, 1)
            return 0
        lax.fori_loop(0, (NCH + 2) // 2, pair, 0)
        wait_scatter((NCH - 1) % 2)

        plsc.subcore_barrier()
        pltpu.sync_copy(s_sh.at[tile_rows], out_hbm.at[cid, tile_rows])

    return k(a_t, b_t, c_e, src_p, dst_p, zeros_s)


def kernel(node_features, edge_features, edge_index, batch, params):
    p = params
    src_p = jnp.pad(edge_index[0], (0, EPAD - E))
    dst_p = jnp.pad(edge_index[1], (0, EPAD - E), constant_values=N)
    nf = jnp.pad(node_features, ((0, NPAD - N), (0, 0)))
    ef = jnp.pad(edge_features, ((0, EPAD - E), (0, 0)))
    bf = jnp.pad(batch.astype(_f32), (0, NPAD - N),
                 constant_values=float(G)).reshape(NPAD, 1)

    enW1, enb1, enW2, enb2 = p['enc_node']
    eeW1, eeb1, eeW2, eeb2 = p['enc_edge']
    wc = jnp.concatenate([lp['msg'][0][2 * H:3 * H, :] for lp in p['layers']], axis=1)
    bc = jnp.concatenate([lp['msg'][1] for lp in p['layers']]).reshape(1, 4 * H)

    cs = _enc_edge(ef, eeW1, eeb1.reshape(1, H), eeW2, eeb2.reshape(1, H), wc, bc)
    hv = _mlp_rows(nf, enW1, enb1.reshape(1, H), enW2, enb2.reshape(1, H), H)

    ones_rows = jnp.zeros((256, 16), _f32).at[:, 0].set(1.0)
    cnt2 = _sc_counts(dst_p, ones_rows, jnp.zeros((NPAD, 16), _f32))
    cnt0, cnt1 = cnt2[0], cnt2[1]

    zeros_s = jnp.zeros((NPAD, H), _f32)
    for l, lp in enumerate(p['layers']):
        mW1, mb1, mW2, mb2 = lp['msg']
        uW1, ub1, uW2, ub2 = lp['upd']
        wab = jnp.concatenate([mW1[:H], mW1[H:2 * H]], axis=1)
        a_t, b_t = _ab(hv, wab)
        s2 = _sc_edge(a_t, b_t, cs[l], src_p, dst_p, zeros_s)
        hv = _update(s2[0], s2[1], cnt0, cnt1, hv, mW2, mb2.reshape(1, H),
                     uW1[:H], uW1[H:], ub1.reshape(1, H), uW2, ub2.reshape(1, H))

    dW1, db1, dW2, db2 = p['dec']
    node_pred = _mlp_rows(hv, dW1, db1.reshape(1, H), dW2,
                          db2.reshape(1, NODE_OUT), NODE_OUT)[:N]

    rW1, rb1, rW2, rb2 = p['ro']
    global_pred = _readout(hv, bf, rW1, rb1.reshape(1, H), rW2, rb2.reshape(1, 1))
    return node_pred, global_pred


# speedup vs baseline: 1.1635x; 1.0011x over previous
import functools
import jax
import jax.numpy as jnp
from jax import lax
from jax.experimental import pallas as pl
from jax.experimental.pallas import tpu as pltpu
from jax.experimental.pallas import tpu_sc as plsc

N = 10000
E = 320000
H = 64
G = 16
NODE_OUT = 3

NC = 2
NS = 16
NW = NC * NS

NPAD = 10240
EPAD = 327680
EW = EPAD // NW
CHUNK = 128
NCH = EW // CHUNK
ROWS_PT = NPAD // NS

BN = 1024
BE = 2048

_f32 = jnp.float32


def _encedge_body(ef, w1, b1, w2, b2, wc, bc, c0, c1, c2, c3):
    h = jnp.maximum(jnp.dot(ef[...], w1[...], preferred_element_type=_f32) + b1[...], 0.0)
    he = jnp.dot(h, w2[...], preferred_element_type=_f32) + b2[...]
    cf = jnp.dot(he, wc[...], preferred_element_type=_f32) + bc[...]
    c0[...] = cf[:, 0:64]
    c1[...] = cf[:, 64:128]
    c2[...] = cf[:, 128:192]
    c3[...] = cf[:, 192:256]


def _enc_edge(ef, w1, b1, w2, b2, wc, bc):
    full = lambda s: pl.BlockSpec(s, lambda i: (0, 0))
    out = jax.ShapeDtypeStruct((EPAD, H), _f32)
    return pl.pallas_call(
        _encedge_body,
        grid=(EPAD // BE,),
        in_specs=[
            pl.BlockSpec((BE, 16), lambda i: (i, 0)),
            full((16, H)), full((1, H)), full((H, H)), full((1, H)),
            full((H, 4 * H)), full((1, 4 * H)),
        ],
        out_specs=[pl.BlockSpec((BE, H), lambda i: (i, 0))] * 4,
        out_shape=[out, out, out, out],
    )(ef, w1, b1, w2, b2, wc, bc)


def _mlp_body(x, w1, b1, w2, b2, o):
    h = jnp.maximum(jnp.dot(x[...], w1[...], preferred_element_type=_f32) + b1[...], 0.0)
    o[...] = jnp.dot(h, w2[...], preferred_element_type=_f32) + b2[...]


def _mlp_rows(x, w1, b1, w2, b2, dout):
    m, din = x.shape
    full = lambda s: pl.BlockSpec(s, lambda i: (0, 0))
    return pl.pallas_call(
        _mlp_body,
        grid=(m // BN,),
        in_specs=[
            pl.BlockSpec((BN, din), lambda i: (i, 0)),
            full((din, H)), full((1, H)), full((H, dout)), full((1, dout)),
        ],
        out_specs=pl.BlockSpec((BN, dout), lambda i: (i, 0)),
        out_shape=jax.ShapeDtypeStruct((m, dout), _f32),
    )(x, w1, b1, w2, b2)


def _ab_body(hv, wab, a, b):
    ab = jnp.dot(hv[...], wab[...], preferred_element_type=_f32)
    a[...] = ab[:, :H]
    b[...] = ab[:, H:]


def _ab(hv, wab):
    out = jax.ShapeDtypeStruct((NPAD, H), _f32)
    return pl.pallas_call(
        _ab_body,
        grid=(NPAD // BN,),
        in_specs=[
            pl.BlockSpec((BN, H), lambda i: (i, 0)),
            pl.BlockSpec((H, 2 * H), lambda i: (0, 0)),
        ],
        out_specs=[pl.BlockSpec((BN, H), lambda i: (i, 0))] * 2,
        out_shape=[out, out],
    )(hv, wab)


def _update_body(s0, s1, c0, c1, hv, wm2, bm2, wu1a, wu1b, bu1, wu2, bu2, o):
    cnt = c0[:, :1] + c1[:, :1]
    s = s0[...] + s1[...]
    aggsum = jnp.dot(s, wm2[...], preferred_element_type=_f32) + cnt * bm2[...]
    agg = aggsum / jnp.clip(cnt, 1.0, None)
    h = jnp.maximum(
        jnp.dot(hv[...], wu1a[...], preferred_element_type=_f32)
        + jnp.dot(agg, wu1b[...], preferred_element_type=_f32) + bu1[...], 0.0)
    o[...] = jnp.dot(h, wu2[...], preferred_element_type=_f32) + bu2[...]


def _update(s0, s1, c0, c1, hv, wm2, bm2, wu1a, wu1b, bu1, wu2, bu2):
    full = lambda s: pl.BlockSpec(s, lambda i: (0, 0))
    row = lambda d: pl.BlockSpec((BN, d), lambda i: (i, 0))
    return pl.pallas_call(
        _update_body,
        grid=(NPAD // BN,),
        in_specs=[
            row(H), row(H), row(16), row(16), row(H),
            full((H, H)), full((1, H)), full((H, H)), full((H, H)),
            full((1, H)), full((H, H)), full((1, H)),
        ],
        out_specs=row(H),
        out_shape=jax.ShapeDtypeStruct((NPAD, H), _f32),
    )(s0, s1, c0, c1, hv, wm2, bm2, wu1a, wu1b, bu1, wu2, bu2)


def _readout_body(hv, bf, w1, b1, w2, b2, o, ssum, scnt):
    i = pl.program_id(0)

    @pl.when(i == 0)
    def _():
        ssum[...] = jnp.zeros_like(ssum)
        scnt[...] = jnp.zeros_like(scnt)

    gids = lax.broadcasted_iota(jnp.int32, (1, G), 1).astype(_f32)
    onehot = (bf[...] == gids).astype(_f32)
    dn = (((0,), (0,)), ((), ()))
    ssum[...] += lax.dot_general(onehot, hv[...], dn, preferred_element_type=_f32)
    scnt[...] += lax.dot_general(onehot, jnp.ones((BN, 1), _f32), dn,
                                 preferred_element_type=_f32)

    @pl.when(i == pl.num_programs(0) - 1)
    def _():
        pooled = ssum[...] / jnp.clip(scnt[...], 1.0, None)
        h = jnp.maximum(jnp.dot(pooled, w1[...], preferred_element_type=_f32) + b1[...], 0.0)
        o[...] = jnp.dot(h, w2[...], preferred_element_type=_f32) + b2[...]


def _readout(hv, bf, w1, b1, w2, b2):
    full = lambda s: pl.BlockSpec(s, lambda i: (0, 0))
    return pl.pallas_call(
        _readout_body,
        grid=(NPAD // BN,),
        in_specs=[
            pl.BlockSpec((BN, H), lambda i: (i, 0)),
            pl.BlockSpec((BN, 1), lambda i: (i, 0)),
            full((H, H)), full((1, H)), full((H, 1)), full((1, 1)),
        ],
        out_specs=full((G, 1)),
        out_shape=jax.ShapeDtypeStruct((G, 1), _f32),
        scratch_shapes=[pltpu.VMEM((G, H), _f32), pltpu.VMEM((G, 1), _f32)],
    )(hv, bf, w1, b1, w2, b2)


_MESH = dict(core_axis_name="c", subcore_axis_name="s")


def _sc_counts(dst_p, ones_rows, zeros_cnt):
    CHC = 256
    nch = EW // CHC

    @functools.partial(
        pl.kernel,
        out_type=jax.ShapeDtypeStruct((NC, NPAD, 16), _f32),
        mesh=plsc.VectorSubcoreMesh(**_MESH),
        compiler_params=pltpu.CompilerParams(use_tc_tiling_on_sc=False),
        scratch_types=[
            pltpu.VMEM((256,), jnp.int32),
            pltpu.VMEM((256, 16), _f32),
            pltpu.VMEM_SHARED((NPAD, 16), _f32),
        ],
    )
    def k(dst_hbm, ones_hbm, zero_hbm, out_hbm, idx_v, ones_v, cnt_sh):
        cid = lax.axis_index("c")
        sid = lax.axis_index("s")
        wid = sid * NC + cid

        @pl.when(sid == 0)
        def _():
            pltpu.sync_copy(zero_hbm, cnt_sh)
        pltpu.sync_copy(ones_hbm, ones_v)
        plsc.subcore_barrier()

        def body(j, _):
            base = wid * EW + j * CHC
            pltpu.sync_copy(dst_hbm.at[pl.ds(base, CHC)], idx_v)
            pltpu.sync_copy(ones_v, cnt_sh.at[idx_v], add=True)
            return 0
        lax.fori_loop(0, nch, body, 0)
        plsc.subcore_barrier()
        pltpu.sync_copy(cnt_sh.at[pl.ds(sid * ROWS_PT, ROWS_PT)],
                        out_hbm.at[cid, pl.ds(sid * ROWS_PT, ROWS_PT)])

    return k(dst_p, ones_rows, zeros_cnt)


def _sc_edge(a_t, b_t, c_e, src_p, dst_p, zeros_s):
    @functools.partial(
        pl.kernel,
        out_type=jax.ShapeDtypeStruct((NC, NPAD, H), _f32),
        mesh=plsc.VectorSubcoreMesh(**_MESH),
        compiler_params=pltpu.CompilerParams(use_tc_tiling_on_sc=False),
        scratch_types=[
            pltpu.VMEM((CHUNK,), jnp.int32),
            pltpu.VMEM((CHUNK,), jnp.int32),
            pltpu.VMEM((CHUNK,), jnp.int32),
            pltpu.VMEM((CHUNK,), jnp.int32),
            pltpu.VMEM((CHUNK,), jnp.int32),
            pltpu.VMEM((CHUNK,), jnp.int32),
            pltpu.VMEM((CHUNK, H), _f32),
            pltpu.VMEM((CHUNK, H), _f32),
            pltpu.VMEM((CHUNK, H), _f32),
            pltpu.VMEM((CHUNK, H), _f32),
            pltpu.VMEM((CHUNK, H), _f32),
            pltpu.VMEM((CHUNK, H), _f32),
            pltpu.VMEM((CHUNK, H), _f32),
            pltpu.VMEM((CHUNK, H), _f32),
            pltpu.SemaphoreType.DMA,
            pltpu.SemaphoreType.DMA,
            pltpu.SemaphoreType.DMA,
            pltpu.SemaphoreType.DMA,
            pltpu.SemaphoreType.DMA,
            pltpu.SemaphoreType.DMA,
            pltpu.VMEM_SHARED((NPAD, H), _f32),
        ],
    )
    def k(a_hbm, b_hbm, c_hbm, src_hbm, dst_hbm, zero_hbm, out_hbm,
          is0, is1, id0, id1, ic0, ic1, ba0, ba1, bb0, bb1, bc0, bc1,
          bh0, bh1, si0, si1, sg0, sg1, ss0, ss1, s_sh):
        IS, ID, IC = [is0, is1], [id0, id1], [ic0, ic1]
        BA, BB, BC, BH = [ba0, ba1], [bb0, bb1], [bc0, bc1], [bh0, bh1]
        SI, SG, SS = [si0, si1], [sg0, sg1], [ss0, ss1]

        cid = lax.axis_index("c")
        sid = lax.axis_index("s")
        wid = sid * NC + cid
        base0 = wid * EW
        tile_rows = pl.ds(sid * ROWS_PT, ROWS_PT)

        @pl.when(sid == 0)
        def _():
            pltpu.sync_copy(zero_hbm, s_sh)
        plsc.subcore_barrier()

        def start_idx(s, j):
            base = base0 + j * CHUNK
            pltpu.async_copy(src_hbm.at[pl.ds(base, CHUNK)], IS[s], SI[s])
            pltpu.async_copy(dst_hbm.at[pl.ds(base, CHUNK)], ID[s], SI[s])
            pltpu.async_copy(c_hbm.at[pl.ds(base, CHUNK)], BC[s], SI[s])

        def wait_idx(s):
            pltpu.make_async_copy(src_hbm.at[pl.ds(0, CHUNK)], IS[s], SI[s]).wait()
            pltpu.make_async_copy(dst_hbm.at[pl.ds(0, CHUNK)], ID[s], SI[s]).wait()
            pltpu.make_async_copy(c_hbm.at[pl.ds(0, CHUNK)], BC[s], SI[s]).wait()

        def start_gather(s):
            pltpu.async_copy(a_hbm.at[IS[s]], BA[s], SG[s])
            pltpu.async_copy(b_hbm.at[ID[s]], BB[s], SG[s])

        def wait_gather(s):
            pltpu.make_async_copy(a_hbm.at[IS[s]], BA[s], SG[s]).wait()
            pltpu.make_async_copy(b_hbm.at[ID[s]], BB[s], SG[s]).wait()

        def start_scatter(s):
            pltpu.async_copy(BH[s], s_sh.at[IC[s]], SS[s], add=True)

        def wait_scatter(s):
            pltpu.make_async_copy(BH[s], s_sh.at[IC[s]], SS[s]).wait()

        def compute(s):
            a_, b_, c_, h_ = BA[s], BB[s], BC[s], BH[s]

            @plsc.parallel_loop(0, CHUNK, unroll=4)
            def _(r):
                for c4 in range(H // 16):
                    sl = pl.ds(c4 * 16, 16)
                    h_[r, sl] = jnp.maximum(a_[r, sl] + b_[r, sl] + c_[r, sl], 0.0)

        def phase(j, s):
            o = 1 - s

            @pl.when(j >= 3)
            def _():
                wait_scatter(o)

            @pl.when(j < NCH)
            def _():
                wait_idx(s)
                start_gather(s)

            @pl.when(jnp.logical_and(j >= 1, j <= NCH))
            def _():
                wait_gather(o)
                for q in range(CHUNK // 16):
                    qs = pl.ds(q * 16, 16)
                    IC[o][qs] = ID[o][qs]
                compute(o)
                start_scatter(o)

            @pl.when(j + 1 < NCH)
            def _():
                start_idx(o, j + 1)

        start_idx(0, 0)

        def pair(i, _):
            j = 2 * i
            phase(j, 0)
            phase(j + 1, 1)
            return 0
        lax.fori_loop(0, (NCH + 2) // 2, pair, 0)
        wait_scatter((NCH - 1) % 2)

        plsc.subcore_barrier()
        pltpu.sync_copy(s_sh.at[tile_rows], out_hbm.at[cid, tile_rows])

    return k(a_t, b_t, c_e, src_p, dst_p, zeros_s)


def kernel(node_features, edge_features, edge_index, batch, params):
    p = params
    src_p = jnp.pad(edge_index[0], (0, EPAD - E))
    dst_p = jnp.pad(edge_index[1], (0, EPAD - E), constant_values=N)
    nf = jnp.pad(node_features, ((0, NPAD - N), (0, 0)))
    ef = jnp.pad(edge_features, ((0, EPAD - E), (0, 0)))
    bf = jnp.pad(batch.astype(_f32), (0, NPAD - N),
                 constant_values=float(G)).reshape(NPAD, 1)

    enW1, enb1, enW2, enb2 = p['enc_node']
    eeW1, eeb1, eeW2, eeb2 = p['enc_edge']
    wc = jnp.concatenate([lp['msg'][0][2 * H:3 * H, :] for lp in p['layers']], axis=1)
    bc = jnp.concatenate([lp['msg'][1] for lp in p['layers']]).reshape(1, 4 * H)

    cs = _enc_edge(ef, eeW1, eeb1.reshape(1, H), eeW2, eeb2.reshape(1, H), wc, bc)
    hv = _mlp_rows(nf, enW1, enb1.reshape(1, H), enW2, enb2.reshape(1, H), H)

    ones_rows = jnp.zeros((256, 16), _f32).at[:, 0].set(1.0)
    cnt2 = _sc_counts(dst_p, ones_rows, jnp.zeros((NPAD, 16), _f32))
    cnt0, cnt1 = cnt2[0], cnt2[1]

    zeros_s = jnp.zeros((NPAD, H), _f32)
    for l, lp in enumerate(p['layers']):
        mW1, mb1, mW2, mb2 = lp['msg']
        uW1, ub1, uW2, ub2 = lp['upd']
        wab = jnp.concatenate([mW1[:H], mW1[H:2 * H]], axis=1)
        a_t, b_t = _ab(hv, wab)
        s2 = _sc_edge(a_t, b_t, cs[l], src_p, dst_p, zeros_s)
        hv = _update(s2[0], s2[1], cnt0, cnt1, hv, mW2, mb2.reshape(1, H),
                     uW1[:H], uW1[H:], ub1.reshape(1, H), uW2, ub2.reshape(1, H))

    dW1, db1, dW2, db2 = p['dec']
    node_pred = _mlp_rows(hv, dW1, db1.reshape(1, H), dW2,
                          db2.reshape(1, NODE_OUT), NODE_OUT)[:N]

    rW1, rb1, rW2, rb2 = p['ro']
    global_pred = _readout(hv, bf, rW1, rb1.reshape(1, H), rW2, rb2.reshape(1, 1))
    return node_pred, global_pred
